# baseline (speedup 1.0000x reference)
import numpy as np

import jax
import jax.numpy as jnp
from jax.experimental import pallas as pl
from jax.experimental.pallas import tpu as pltpu
from jax.sharding import Mesh, PartitionSpec as P

try:
    from jax import shard_map as _shard_map

    def _smap(f, mesh, in_specs, out_specs):
        return _shard_map(f, mesh=mesh, in_specs=in_specs,
                          out_specs=out_specs, check_vma=False)
except ImportError:
    from jax.experimental.shard_map import shard_map as _shard_map_legacy

    def _smap(f, mesh, in_specs, out_specs):
        return _shard_map_legacy(f, mesh=mesh, in_specs=in_specs,
                                 out_specs=out_specs, check_rep=False)

_BB = 64

_F32 = jnp.float32


def _sel1():
    S = np.zeros((3, 32, 2, 16), np.float32)
    for dx in range(3):
        for p in range(2):
            for j in range(16):
                ow = 2 * j + p
                if ow < 26:
                    S[dx, ow + dx, p, j] = 1.0
    return S


def _sel2():
    S = np.zeros((3, 16, 2, 8), np.float32)
    for dx in range(3):
        for p in range(2):
            for j in range(8):
                ow = 2 * j + p
                if ow < 11:
                    S[dx, ow + dx, p, j] = 1.0
    return S


def _sel3():
    S = np.zeros((3, 8, 2), np.float32)
    for dx in range(3):
        for p in range(2):
            S[dx, p + dx, p] = 1.0
    return S


_S1 = _sel1()
_S2 = _sel2()
_S3 = _sel3()
_M1 = np.zeros((2, 16), np.float32)
for _p in range(2):
    for _j in range(16):
        if 2 * _j + _p < 26:
            _M1[_p, _j] = 1.0
_M2 = np.zeros((2, 8), np.float32)
for _p in range(2):
    for _j in range(8):
        if 2 * _j + _p < 11:
            _M2[_p, _j] = 1.0


def _body(x_ref, t1_ref, b1_ref, t2_ref, b2_ref, t3_ref, b3_ref,
          hw1_ref, hb1_ref, hw2_ref, hb2_ref, out_ref,
          p1, a1, q1, p2, a2, q2, p3, a3):
    BB = _BB
    f32 = _F32

    vx = x_ref[...].astype(f32)
    for dy in range(3):
        p1[:, 0:26, dy * 32:dy * 32 + 28] = vx[:, dy:dy + 26, :]
        p1[:, 0:26, dy * 32 + 28:dy * 32 + 32] = jnp.zeros((BB, 26, 4), f32)
    o1 = jnp.dot(p1[...].reshape(BB * 32, 96), t1_ref[...],
                 preferred_element_type=f32)
    a1[...] = jnp.maximum(o1 + b1_ref[...], 0.0).reshape(BB, 32, 1024)

    for s in range(13):
        m = jnp.maximum(a1[:, 2 * s, :], a1[:, 2 * s + 1, :])
        q1[:, s, :] = jnp.maximum(m[:, 0:512], m[:, 512:1024])

    for dy in range(3):
        p2[:, 0:11, dy * 512:(dy + 1) * 512] = q1[:, dy:dy + 11, :]
    o2 = jnp.dot(p2[...].reshape(BB * 16, 1536), t2_ref[...],
                 preferred_element_type=f32)
    a2[...] = jnp.maximum(o2 + b2_ref[...], 0.0).reshape(BB, 16, 1024)

    for s in range(5):
        m2 = jnp.maximum(a2[:, 2 * s, :], a2[:, 2 * s + 1, :])
        q2[:, s, :] = jnp.maximum(m2[:, 0:512], m2[:, 512:1024])

    for dy in range(3):
        p3[:, 0:2, dy * 512:(dy + 1) * 512] = q2[:, dy:dy + 2, :]
    o3 = jnp.dot(p3[...].reshape(BB * 8, 1536), t3_ref[...],
                 preferred_element_type=f32)
    a3[...] = jnp.maximum(o3 + b3_ref[...], 0.0).reshape(BB, 8, 256)

    mm = jnp.maximum(a3[:, 0, :], a3[:, 1, :])
    feat = jnp.maximum(mm[:, 0:128], mm[:, 128:256])
    h = jnp.maximum(
        jnp.dot(feat, hw1_ref[...], preferred_element_type=f32) + hb1_ref[...],
        0.0)
    out_ref[...] = (jnp.dot(h, hw2_ref[...], preferred_element_type=f32)
                    + hb2_ref[...])


def _forward_block(x3, cw1, cb1, cw2, cb2, cw3, cb3, hw1, hb1, hw2, hb2):
    Bs = x3.shape[0]
    BB = _BB
    const2 = lambda b: (0, 0)

    t1 = jnp.einsum('yxc,xwpj->ywpjc', cw1[:, :, 0, :].astype(_F32),
                    _S1).reshape(96, 1024)
    t2 = jnp.einsum('yxio,xwpj->ywipjo', cw2.astype(_F32),
                    _S2).reshape(1536, 1024)
    t3f = jnp.einsum('yxio,xwp->ywipo', cw3.astype(_F32), _S3)
    t3 = jnp.pad(t3f, ((0, 0), (0, 0), (0, 0), (0, 0), (0, 96))
                 ).reshape(1536, 256)

    b1v = (jnp.asarray(_M1)[:, :, None] * cb1.astype(_F32)).reshape(1, 1024)
    b2v = (jnp.asarray(_M2)[:, :, None] * cb2.astype(_F32)).reshape(1, 1024)
    b3v = jnp.tile(jnp.pad(cb3.astype(_F32), (0, 96)), 2).reshape(1, 256)

    hw1p = jnp.pad(hw1.astype(_F32), ((0, 96), (0, 118)))
    hb1p = jnp.pad(hb1.astype(_F32), (0, 118)).reshape(1, 128)
    hw2p = jnp.pad(hw2.astype(_F32), ((0, 118), (0, 118)))
    hb2p = jnp.pad(hb2.astype(_F32), (0, 118)).reshape(1, 128)

    return pl.pallas_call(
        _body,
        out_shape=jax.ShapeDtypeStruct((Bs, 128), _F32),
        grid=(Bs // BB,),
        in_specs=[
            pl.BlockSpec((BB, 28, 28), lambda b: (b, 0, 0)),

            pl.BlockSpec((96, 1024), const2),
            pl.BlockSpec((1, 1024), const2),
            pl.BlockSpec((1536, 1024), const2),
            pl.BlockSpec((1, 1024), const2),
            pl.BlockSpec((1536, 256), const2),
            pl.BlockSpec((1, 256), const2),
            pl.BlockSpec((128, 128), const2),
            pl.BlockSpec((1, 128), const2),
            pl.BlockSpec((128, 128), const2),
            pl.BlockSpec((1, 128), const2),
        ],
        out_specs=pl.BlockSpec((BB, 128), lambda b: (b, 0)),
        scratch_shapes=[
            pltpu.VMEM((BB, 32, 96), _F32),
            pltpu.VMEM((BB, 32, 1024), _F32),
            pltpu.VMEM((BB, 16, 512), _F32),
            pltpu.VMEM((BB, 16, 1536), _F32),
            pltpu.VMEM((BB, 16, 1024), _F32),
            pltpu.VMEM((BB, 8, 512), _F32),
            pltpu.VMEM((BB, 8, 1536), _F32),
            pltpu.VMEM((BB, 8, 256), _F32),
        ],
        compiler_params=pltpu.CompilerParams(
            dimension_semantics=("arbitrary",),
            vmem_limit_bytes=100 * 1024 * 1024,
        ),
    )(x3, t1, b1v, t2, b2v, t3, b3v, hw1p, hb1p, hw2p, hb2p)


def kernel(x_nchw, cw1, cb1, cw2, cb2, cw3, cb3, hw1, hb1, hw2, hb2):
    B = x_nchw.shape[0]
    x3 = x_nchw.reshape(B, 28, 28).astype(jnp.bfloat16)
    BB = _BB

    tpus = [d for d in jax.devices() if d.platform == "tpu"]
    ndev = 2 if len(tpus) >= 2 else 1

    pad_b = (-B) % (BB * ndev)
    if pad_b:
        x3 = jnp.pad(x3, ((0, pad_b), (0, 0), (0, 0)))

    args = (x3, cw1, cb1, cw2, cb2, cw3, cb3, hw1, hb1, hw2, hb2)

    if ndev == 2:
        mesh = Mesh(np.array(tpus[:2]), ("d",))
        fwd = _smap(_forward_block, mesh,
                    (P("d"),) + (P(),) * 10,
                    P("d", None))
        out = fwd(*args)
    else:
        out = _forward_block(*args)

    return out[:B, :10]


# speedup vs baseline: 1.6165x; 1.6165x over previous
import numpy as np

import jax
import jax.numpy as jnp
from jax.experimental import pallas as pl
from jax.experimental.pallas import tpu as pltpu
_BB = 64

_F32 = jnp.float32


def _sel1():
    S = np.zeros((3, 32, 2, 16), np.float32)
    for dx in range(3):
        for p in range(2):
            for j in range(16):
                ow = 2 * j + p
                if ow < 26:
                    S[dx, ow + dx, p, j] = 1.0
    return S


def _sel2():
    S = np.zeros((3, 16, 2, 6), np.float32)
    for dx in range(3):
        for p in range(2):
            for j in range(6):
                ow = 2 * j + p
                if ow < 11:
                    S[dx, ow + dx, p, j] = 1.0
    return S


def _sel3():
    S = np.zeros((3, 6, 2), np.float32)
    for dx in range(3):
        for p in range(2):
            S[dx, p + dx, p] = 1.0
    return S


_S1 = _sel1()
_S2 = _sel2()
_S3 = _sel3()
_M1 = np.zeros((2, 16), np.float32)
for _p in range(2):
    for _j in range(16):
        if 2 * _j + _p < 26:
            _M1[_p, _j] = 1.0
_M2 = np.zeros((2, 6), np.float32)
for _p in range(2):
    for _j in range(6):
        if 2 * _j + _p < 11:
            _M2[_p, _j] = 1.0


def _body(x_ref, t1_ref, b1_ref, t2_ref, b2_ref, t3_ref, b3_ref,
          hw1_ref, hb1_ref, hw2_ref, hb2_ref, out_ref,
          p1, a1, p2, a2, p3, a3):
    BB = _BB
    f32 = _F32

    vx = x_ref[...].astype(f32)
    for dy in range(3):
        p1[0:26, :, dy * 32:dy * 32 + 28] = vx[dy:dy + 26, :, :]
        p1[0:26, :, dy * 32 + 28:dy * 32 + 32] = jnp.zeros((26, BB, 4), f32)
    o1 = jnp.dot(p1[...].reshape(26 * BB, 96), t1_ref[...],
                 preferred_element_type=f32)
    a1[...] = jnp.maximum(o1 + b1_ref[...], 0.0).reshape(26, BB, 1024)

    for s in range(13):
        m = jnp.maximum(a1[2 * s], a1[2 * s + 1])
        v = jnp.maximum(m[:, 0:512], m[:, 512:1024])
        for dy in range(3):
            s2 = s - dy
            if 0 <= s2 <= 10:
                p2[s2, :, dy * 512:(dy + 1) * 512] = v

    o2 = jnp.dot(p2[...].reshape(11 * BB, 1536), t2_ref[...],
                 preferred_element_type=f32)
    a2[...] = jnp.maximum(o2 + b2_ref[...], 0.0).reshape(11, BB, 768)

    for s in range(4):
        m2 = jnp.maximum(a2[2 * s], a2[2 * s + 1])
        v2 = jnp.maximum(m2[:, 0:384], m2[:, 384:768])
        for dy in range(3):
            s3 = s - dy
            if 0 <= s3 <= 1:
                p3[s3, :, dy * 384:(dy + 1) * 384] = v2

    o3 = jnp.dot(p3[...].reshape(2 * BB, 1152), t3_ref[...],
                 preferred_element_type=f32)
    a3[...] = jnp.maximum(o3 + b3_ref[...], 0.0).reshape(2, BB, 256)

    mm = jnp.maximum(a3[0], a3[1])
    feat = jnp.maximum(mm[:, 0:128], mm[:, 128:256])
    h = jnp.maximum(
        jnp.dot(feat, hw1_ref[...], preferred_element_type=f32) + hb1_ref[...],
        0.0)
    out_ref[...] = (jnp.dot(h, hw2_ref[...], preferred_element_type=f32)
                    + hb2_ref[...])


def _forward_block(xt, cw1, cb1, cw2, cb2, cw3, cb3, hw1, hb1, hw2, hb2):
    Bs = xt.shape[1]
    BB = _BB
    const2 = lambda b: (0, 0)
    const3 = lambda b: (0, 0, 0)

    t1 = jnp.einsum('yxc,xwpj->ywpjc', cw1[:, :, 0, :].astype(_F32),
                    _S1).reshape(96, 1024)
    t2 = jnp.einsum('yxio,xwpj->ywipjo', cw2.astype(_F32),
                    _S2).reshape(1536, 768)
    t3f = jnp.einsum('yxio,xwp->ywipo', cw3.astype(_F32), _S3)
    t3 = jnp.pad(t3f, ((0, 0), (0, 0), (0, 0), (0, 0), (0, 96))
                 ).reshape(1152, 256)

    b1v = (jnp.asarray(_M1)[:, :, None] * cb1.astype(_F32)).reshape(1, 1024)
    b2v = (jnp.asarray(_M2)[:, :, None] * cb2.astype(_F32)).reshape(1, 768)
    b3v = jnp.tile(jnp.pad(cb3.astype(_F32), (0, 96)), 2).reshape(1, 256)

    hw1p = jnp.pad(hw1.astype(_F32), ((0, 96), (0, 118)))
    hb1p = jnp.pad(hb1.astype(_F32), (0, 118)).reshape(1, 128)
    hw2p = jnp.pad(hw2.astype(_F32), ((0, 118), (0, 118)))
    hb2p = jnp.pad(hb2.astype(_F32), (0, 118)).reshape(1, 128)

    return pl.pallas_call(
        _body,
        out_shape=jax.ShapeDtypeStruct((Bs, 128), _F32),
        grid=(Bs // BB,),
        in_specs=[
            pl.BlockSpec((28, BB, 28), lambda b: (0, b, 0)),
            pl.BlockSpec((96, 1024), const2),
            pl.BlockSpec((1, 1024), const2),
            pl.BlockSpec((1536, 768), const2),
            pl.BlockSpec((1, 768), const2),
            pl.BlockSpec((1152, 256), const2),
            pl.BlockSpec((1, 256), const2),
            pl.BlockSpec((128, 128), const2),
            pl.BlockSpec((1, 128), const2),
            pl.BlockSpec((128, 128), const2),
            pl.BlockSpec((1, 128), const2),
        ],
        out_specs=pl.BlockSpec((BB, 128), lambda b: (b, 0)),
        scratch_shapes=[
            pltpu.VMEM((26, BB, 96), _F32),
            pltpu.VMEM((26, BB, 1024), _F32),
            pltpu.VMEM((11, BB, 1536), _F32),
            pltpu.VMEM((11, BB, 768), _F32),
            pltpu.VMEM((2, BB, 1152), _F32),
            pltpu.VMEM((2, BB, 256), _F32),
        ],
        compiler_params=pltpu.CompilerParams(
            dimension_semantics=("arbitrary",),
            vmem_limit_bytes=100 * 1024 * 1024,
        ),
    )(xt, t1, b1v, t2, b2v, t3, b3v, hw1p, hb1p, hw2p, hb2p)


def _shard_fn(x3, cw1, cb1, cw2, cb2, cw3, cb3, hw1, hb1, hw2, hb2):
    xt = jnp.transpose(x3, (1, 0, 2))
    return _forward_block(xt, cw1, cb1, cw2, cb2, cw3, cb3,
                          hw1, hb1, hw2, hb2)


def kernel(x_nchw, cw1, cb1, cw2, cb2, cw3, cb3, hw1, hb1, hw2, hb2):
    B = x_nchw.shape[0]
    x3 = x_nchw.reshape(B, 28, 28).astype(jnp.bfloat16)
    BB = _BB

    pad_b = (-B) % BB
    if pad_b:
        x3 = jnp.pad(x3, ((0, pad_b), (0, 0), (0, 0)))

    out = _shard_fn(x3, cw1, cb1, cw2, cb2, cw3, cb3, hw1, hb1, hw2, hb2)

    return out[:B, :10]


# speedup vs baseline: 1.7249x; 1.0671x over previous
import numpy as np

import jax
import jax.numpy as jnp
from jax.experimental import pallas as pl
from jax.experimental.pallas import tpu as pltpu
_BB = 128

_F32 = jnp.float32


def _sel1():
    S = np.zeros((3, 32, 2, 16), np.float32)
    for dx in range(3):
        for p in range(2):
            for j in range(16):
                ow = 2 * j + p
                if ow < 26:
                    S[dx, ow + dx, p, j] = 1.0
    return S


def _sel2():
    S = np.zeros((3, 16, 2, 6), np.float32)
    for dx in range(3):
        for p in range(2):
            for j in range(6):
                ow = 2 * j + p
                if ow < 11:
                    S[dx, ow + dx, p, j] = 1.0
    return S


def _sel3():
    S = np.zeros((3, 6, 2), np.float32)
    for dx in range(3):
        for p in range(2):
            S[dx, p + dx, p] = 1.0
    return S


_S1 = _sel1()
_S2 = _sel2()
_S3 = _sel3()
_M1 = np.zeros((2, 16), np.float32)
for _p in range(2):
    for _j in range(16):
        if 2 * _j + _p < 26:
            _M1[_p, _j] = 1.0
_M2 = np.zeros((2, 6), np.float32)
for _p in range(2):
    for _j in range(6):
        if 2 * _j + _p < 11:
            _M2[_p, _j] = 1.0


def _body(x_ref, t1_ref, b1_ref, t2_ref, b2_ref, t3_ref, b3_ref,
          hw1_ref, hb1_ref, hw2_ref, hb2_ref, out_ref,
          p1, a1, p2, a2, p3, a3):
    BB = _BB
    f32 = _F32

    vx = x_ref[...].astype(f32)
    for dy in range(3):
        p1[0:26, :, dy * 32:dy * 32 + 28] = vx[dy:dy + 26, :, :]
        p1[0:26, :, dy * 32 + 28:dy * 32 + 32] = jnp.zeros((26, BB, 4), f32)
    o1 = jnp.dot(p1[...].reshape(26 * BB, 96), t1_ref[...],
                 preferred_element_type=f32)
    a1[...] = jnp.maximum(o1 + b1_ref[...], 0.0).reshape(26, BB, 1024)

    for s in range(13):
        m = jnp.maximum(a1[2 * s], a1[2 * s + 1])
        v = jnp.maximum(m[:, 0:512], m[:, 512:1024])
        for dy in range(3):
            s2 = s - dy
            if 0 <= s2 <= 10:
                p2[s2, :, dy * 512:(dy + 1) * 512] = v

    o2 = jnp.dot(p2[...].reshape(11 * BB, 1536), t2_ref[...],
                 preferred_element_type=f32)
    a2[...] = jnp.maximum(o2 + b2_ref[...], 0.0).reshape(11, BB, 768)

    for s in range(4):
        m2 = jnp.maximum(a2[2 * s], a2[2 * s + 1])
        v2 = jnp.maximum(m2[:, 0:384], m2[:, 384:768])
        for dy in range(3):
            s3 = s - dy
            if 0 <= s3 <= 1:
                p3[s3, :, dy * 384:(dy + 1) * 384] = v2

    o3 = jnp.dot(p3[...].reshape(2 * BB, 1152), t3_ref[...],
                 preferred_element_type=f32)
    a3[...] = jnp.maximum(o3 + b3_ref[...], 0.0).reshape(2, BB, 256)

    mm = jnp.maximum(a3[0], a3[1])
    feat = jnp.maximum(mm[:, 0:128], mm[:, 128:256])
    h = jnp.maximum(
        jnp.dot(feat, hw1_ref[...], preferred_element_type=f32) + hb1_ref[...],
        0.0)
    out_ref[...] = (jnp.dot(h, hw2_ref[...], preferred_element_type=f32)
                    + hb2_ref[...])


def _forward_block(xt, cw1, cb1, cw2, cb2, cw3, cb3, hw1, hb1, hw2, hb2):
    Bs = xt.shape[1]
    BB = _BB
    const2 = lambda b: (0, 0)
    const3 = lambda b: (0, 0, 0)

    t1 = jnp.einsum('yxc,xwpj->ywpjc', cw1[:, :, 0, :].astype(_F32),
                    _S1).reshape(96, 1024)
    t2 = jnp.einsum('yxio,xwpj->ywipjo', cw2.astype(_F32),
                    _S2).reshape(1536, 768)
    t3f = jnp.einsum('yxio,xwp->ywipo', cw3.astype(_F32), _S3)
    t3 = jnp.pad(t3f, ((0, 0), (0, 0), (0, 0), (0, 0), (0, 96))
                 ).reshape(1152, 256)

    b1v = (jnp.asarray(_M1)[:, :, None] * cb1.astype(_F32)).reshape(1, 1024)
    b2v = (jnp.asarray(_M2)[:, :, None] * cb2.astype(_F32)).reshape(1, 768)
    b3v = jnp.tile(jnp.pad(cb3.astype(_F32), (0, 96)), 2).reshape(1, 256)

    hw1p = jnp.pad(hw1.astype(_F32), ((0, 96), (0, 118)))
    hb1p = jnp.pad(hb1.astype(_F32), (0, 118)).reshape(1, 128)
    hw2p = jnp.pad(hw2.astype(_F32), ((0, 118), (0, 118)))
    hb2p = jnp.pad(hb2.astype(_F32), (0, 118)).reshape(1, 128)

    return pl.pallas_call(
        _body,
        out_shape=jax.ShapeDtypeStruct((Bs, 128), _F32),
        grid=(Bs // BB,),
        in_specs=[
            pl.BlockSpec((28, BB, 28), lambda b: (0, b, 0)),
            pl.BlockSpec((96, 1024), const2),
            pl.BlockSpec((1, 1024), const2),
            pl.BlockSpec((1536, 768), const2),
            pl.BlockSpec((1, 768), const2),
            pl.BlockSpec((1152, 256), const2),
            pl.BlockSpec((1, 256), const2),
            pl.BlockSpec((128, 128), const2),
            pl.BlockSpec((1, 128), const2),
            pl.BlockSpec((128, 128), const2),
            pl.BlockSpec((1, 128), const2),
        ],
        out_specs=pl.BlockSpec((BB, 128), lambda b: (b, 0)),
        scratch_shapes=[
            pltpu.VMEM((26, BB, 96), _F32),
            pltpu.VMEM((26, BB, 1024), _F32),
            pltpu.VMEM((11, BB, 1536), _F32),
            pltpu.VMEM((11, BB, 768), _F32),
            pltpu.VMEM((2, BB, 1152), _F32),
            pltpu.VMEM((2, BB, 256), _F32),
        ],
        compiler_params=pltpu.CompilerParams(
            dimension_semantics=("arbitrary",),
            vmem_limit_bytes=100 * 1024 * 1024,
        ),
    )(xt, t1, b1v, t2, b2v, t3, b3v, hw1p, hb1p, hw2p, hb2p)


def _shard_fn(x3, cw1, cb1, cw2, cb2, cw3, cb3, hw1, hb1, hw2, hb2):
    xt = jnp.transpose(x3, (1, 0, 2))
    return _forward_block(xt, cw1, cb1, cw2, cb2, cw3, cb3,
                          hw1, hb1, hw2, hb2)


def kernel(x_nchw, cw1, cb1, cw2, cb2, cw3, cb3, hw1, hb1, hw2, hb2):
    B = x_nchw.shape[0]
    x3 = x_nchw.reshape(B, 28, 28).astype(jnp.bfloat16)
    BB = _BB

    pad_b = (-B) % BB
    if pad_b:
        x3 = jnp.pad(x3, ((0, pad_b), (0, 0), (0, 0)))

    out = _shard_fn(x3, cw1, cb1, cw2, cb2, cw3, cb3, hw1, hb1, hw2, hb2)

    return out[:B, :10]


# speedup vs baseline: 1.7254x; 1.0003x over previous
import numpy as np

import jax
import jax.numpy as jnp
from jax.experimental import pallas as pl
from jax.experimental.pallas import tpu as pltpu
_BB = 128

_F32 = jnp.float32


def _sel1():
    S = np.zeros((3, 32, 2, 16), np.float32)
    for dx in range(3):
        for p in range(2):
            for j in range(16):
                ow = 2 * j + p
                if ow < 26:
                    S[dx, ow + dx, p, j] = 1.0
    return S


def _sel2():
    S = np.zeros((3, 16, 2, 6), np.float32)
    for dx in range(3):
        for p in range(2):
            for j in range(6):
                ow = 2 * j + p
                if ow < 11:
                    S[dx, ow + dx, p, j] = 1.0
    return S


def _sel3():
    S = np.zeros((3, 6, 2), np.float32)
    for dx in range(3):
        for p in range(2):
            S[dx, p + dx, p] = 1.0
    return S


_S1 = _sel1()
_S2 = _sel2()
_S3 = _sel3()
_M1 = np.zeros((2, 16), np.float32)
for _p in range(2):
    for _j in range(16):
        if 2 * _j + _p < 26:
            _M1[_p, _j] = 1.0
_M2 = np.zeros((2, 6), np.float32)
for _p in range(2):
    for _j in range(6):
        if 2 * _j + _p < 11:
            _M2[_p, _j] = 1.0


def _body(x_ref, t1_ref, b1_ref, t2_ref, b2_ref, t3_ref, b3_ref,
          hw1_ref, hb1_ref, hw2_ref, hb2_ref, out_ref,
          p1, a1, p2, a2, p3, a3):
    BB = _BB
    f32 = _F32

    vx = x_ref[...].astype(f32)
    for dy in range(3):
        p1[0:26, :, dy * 32:dy * 32 + 28] = vx[dy:dy + 26, :, :]
        p1[0:26, :, dy * 32 + 28:dy * 32 + 32] = jnp.zeros((26, BB, 4), f32)
    o1 = jnp.dot(p1[...].reshape(26 * BB, 96), t1_ref[...],
                 preferred_element_type=f32)
    a1[...] = jnp.maximum(o1 + b1_ref[...], 0.0).reshape(26, BB, 1024)

    for s in range(13):
        m = jnp.maximum(a1[2 * s], a1[2 * s + 1])
        v = jnp.maximum(m[:, 0:512], m[:, 512:1024])
        for dy in range(3):
            s2 = s - dy
            if 0 <= s2 <= 10:
                p2[s2, :, dy * 512:(dy + 1) * 512] = v

    o2 = jnp.dot(p2[...].reshape(11 * BB, 1536), t2_ref[...],
                 preferred_element_type=f32)
    a2[...] = jnp.maximum(o2 + b2_ref[...], 0.0).reshape(11, BB, 768)

    for s in range(4):
        m2 = jnp.maximum(a2[2 * s], a2[2 * s + 1])
        v2 = jnp.maximum(m2[:, 0:384], m2[:, 384:768])
        for dy in range(3):
            s3 = s - dy
            if 0 <= s3 <= 1:
                p3[s3, :, dy * 384:(dy + 1) * 384] = v2

    o3 = jnp.dot(p3[...].reshape(2 * BB, 1152), t3_ref[...],
                 preferred_element_type=f32)
    a3[...] = jnp.maximum(o3 + b3_ref[...], 0.0).reshape(2, BB, 256)

    mm = jnp.maximum(a3[0], a3[1])
    feat = jnp.maximum(mm[:, 0:128], mm[:, 128:256])
    h = jnp.maximum(
        jnp.dot(feat, hw1_ref[...], preferred_element_type=f32) + hb1_ref[...],
        0.0)
    logits = (jnp.dot(h, hw2_ref[...], preferred_element_type=f32)
              + hb2_ref[...])
    out_ref[...] = logits[:, 0:10]


def _forward_block(xt, cw1, cb1, cw2, cb2, cw3, cb3, hw1, hb1, hw2, hb2):
    Bs = xt.shape[1]
    BB = _BB
    const2 = lambda b: (0, 0)
    const3 = lambda b: (0, 0, 0)

    t1 = jnp.einsum('yxc,xwpj->ywpjc', cw1[:, :, 0, :].astype(_F32),
                    _S1).reshape(96, 1024)
    t2 = jnp.einsum('yxio,xwpj->ywipjo', cw2.astype(_F32),
                    _S2).reshape(1536, 768)
    t3f = jnp.einsum('yxio,xwp->ywipo', cw3.astype(_F32), _S3)
    t3 = jnp.pad(t3f, ((0, 0), (0, 0), (0, 0), (0, 0), (0, 96))
                 ).reshape(1152, 256)

    b1v = (jnp.asarray(_M1)[:, :, None] * cb1.astype(_F32)).reshape(1, 1024)
    b2v = (jnp.asarray(_M2)[:, :, None] * cb2.astype(_F32)).reshape(1, 768)
    b3v = jnp.tile(jnp.pad(cb3.astype(_F32), (0, 96)), 2).reshape(1, 256)

    hw1p = jnp.pad(hw1.astype(_F32), ((0, 96), (0, 118)))
    hb1p = jnp.pad(hb1.astype(_F32), (0, 118)).reshape(1, 128)
    hw2p = jnp.pad(hw2.astype(_F32), ((0, 118), (0, 118)))
    hb2p = jnp.pad(hb2.astype(_F32), (0, 118)).reshape(1, 128)

    return pl.pallas_call(
        _body,
        out_shape=jax.ShapeDtypeStruct((Bs, 10), _F32),
        grid=(Bs // BB,),
        in_specs=[
            pl.BlockSpec((28, BB, 28), lambda b: (0, b, 0)),
            pl.BlockSpec((96, 1024), const2),
            pl.BlockSpec((1, 1024), const2),
            pl.BlockSpec((1536, 768), const2),
            pl.BlockSpec((1, 768), const2),
            pl.BlockSpec((1152, 256), const2),
            pl.BlockSpec((1, 256), const2),
            pl.BlockSpec((128, 128), const2),
            pl.BlockSpec((1, 128), const2),
            pl.BlockSpec((128, 128), const2),
            pl.BlockSpec((1, 128), const2),
        ],
        out_specs=pl.BlockSpec((BB, 10), lambda b: (b, 0)),
        scratch_shapes=[
            pltpu.VMEM((26, BB, 96), _F32),
            pltpu.VMEM((26, BB, 1024), _F32),
            pltpu.VMEM((11, BB, 1536), _F32),
            pltpu.VMEM((11, BB, 768), _F32),
            pltpu.VMEM((2, BB, 1152), _F32),
            pltpu.VMEM((2, BB, 256), _F32),
        ],
        compiler_params=pltpu.CompilerParams(
            dimension_semantics=("arbitrary",),
            vmem_limit_bytes=100 * 1024 * 1024,
        ),
    )(xt, t1, b1v, t2, b2v, t3, b3v, hw1p, hb1p, hw2p, hb2p)


def _shard_fn(x3, cw1, cb1, cw2, cb2, cw3, cb3, hw1, hb1, hw2, hb2):
    xt = jnp.transpose(x3, (1, 0, 2))
    return _forward_block(xt, cw1, cb1, cw2, cb2, cw3, cb3,
                          hw1, hb1, hw2, hb2)


def kernel(x_nchw, cw1, cb1, cw2, cb2, cw3, cb3, hw1, hb1, hw2, hb2):
    B = x_nchw.shape[0]
    x3 = x_nchw.reshape(B, 28, 28).astype(jnp.bfloat16)
    BB = _BB

    pad_b = (-B) % BB
    if pad_b:
        x3 = jnp.pad(x3, ((0, pad_b), (0, 0), (0, 0)))

    out = _shard_fn(x3, cw1, cb1, cw2, cb2, cw3, cb3, hw1, hb1, hw2, hb2)

    return out[:B] if pad_b else out
